# Initial kernel scaffold; baseline (speedup 1.0000x reference)
import jax
import jax.numpy as jnp
from jax.experimental import pallas as pl


def kernel(points, pts_coors, voxel_feats, voxel2point_inds, voxel_padding):
    raise NotImplementedError("write your pallas kernel here")


# speedup vs baseline: 1.3260x; 1.3260x over previous
import jax
import jax.numpy as jnp
from jax import lax
from jax.experimental import pallas as pl
from jax.experimental.pallas import tpu as pltpu
from jax.experimental.pallas import tpu_sc as plsc

N = 262144
M = 65536
C = 128
NW = 32
CHUNK = N // NW
SUB = 128
NSUB = CHUNK // SUB

VOXEL_SIZE = (1.0, 1.0, 0.08)
PC_MIN = (-50.0, -50.0, -5.0)

_LANES = 16


def _flags_body(pad_ref, feats_ref, flags_ref):
    pad = pad_ref[0, 0]
    flags_ref[...] = jnp.all(feats_ref[...] == pad, axis=1).astype(jnp.int32)


def _compute_flags(pad, voxel_feats):
    BM = 1024
    return pl.pallas_call(
        _flags_body,
        grid=(M // BM,),
        in_specs=[
            pl.BlockSpec(memory_space=pltpu.SMEM),
            pl.BlockSpec((BM, C), lambda i: (i, 0)),
        ],
        out_specs=pl.BlockSpec((BM,), lambda i: (i,)),
        out_shape=jax.ShapeDtypeStruct((M,), jnp.int32),
    )(pad, voxel_feats)


def _maskscan_body(flags_hbm, inds_hbm, mask_hbm, lcsum_hbm, totals_hbm,
                   flags_v, inds_v, mask_v, lcsum_v, tot16_v):
    cid = lax.axis_index("c")
    sid = lax.axis_index("s")
    wid = cid * 16 + sid
    base = wid * CHUNK
    lanes = jnp.arange(_LANES, dtype=jnp.int32)

    pltpu.sync_copy(flags_hbm, flags_v)
    pltpu.sync_copy(inds_hbm.at[pl.ds(base, CHUNK)], inds_v)

    def step(k, carry):
        idx = inds_v[pl.ds(k * _LANES, _LANES)]
        flg = plsc.load_gather(flags_v, [idx >> 7, idx & 127])
        m = 1 - flg
        mask_v[pl.ds(k * _LANES, _LANES)] = m
        lcsum_v[pl.ds(k * _LANES, _LANES)] = plsc.cumsum(m) + carry
        return carry + jnp.sum(m)

    total = lax.fori_loop(0, CHUNK // _LANES, step, jnp.int32(0))

    tot16_v[...] = jnp.where(lanes == 0, total, 0)
    pltpu.sync_copy(mask_v, mask_hbm.at[pl.ds(base, CHUNK)])
    pltpu.sync_copy(lcsum_v, lcsum_hbm.at[pl.ds(base, CHUNK)])
    pltpu.sync_copy(tot16_v, totals_hbm.at[wid])


def _mask_scan(flags, inds):
    mesh = plsc.VectorSubcoreMesh(core_axis_name="c", subcore_axis_name="s")
    return pl.kernel(
        _maskscan_body,
        compiler_params=pltpu.CompilerParams(needs_layout_passes=False, use_tc_tiling_on_sc=False),
        out_type=(
            jax.ShapeDtypeStruct((N,), jnp.int32),
            jax.ShapeDtypeStruct((N,), jnp.int32),
            jax.ShapeDtypeStruct((NW, _LANES), jnp.int32),
        ),
        mesh=mesh,
        scratch_types=[
            pltpu.VMEM((M // 128, 128), jnp.int32),
            pltpu.VMEM((CHUNK,), jnp.int32),
            pltpu.VMEM((CHUNK,), jnp.int32),
            pltpu.VMEM((CHUNK,), jnp.int32),
            pltpu.VMEM((_LANES,), jnp.int32),
        ],
    )(flags, inds)


def _main_body(points_hbm, coors_hbm, vfeats_hbm, inds2_hbm, mask_hbm,
               lcsum_hbm, totals_hbm, out_hbm,
               mask_v, lcsum_v, inds2_v, dest2_v, tot_v,
               f0, f1, p0, p1, q0, q1, o0, o1, g0, g1, s0, s1):
    feats_bufs = (f0, f1)
    pts_bufs = (p0, p1)
    coor_bufs = (q0, q1)
    out_bufs = (o0, o1)
    gsems = (g0, g1)
    ssems = (s0, s1)

    cid = lax.axis_index("c")
    sid = lax.axis_index("s")
    wid = cid * 16 + sid
    base = wid * CHUNK
    lanes = jnp.arange(_LANES, dtype=jnp.int32)
    zeros16 = jnp.zeros((_LANES,), jnp.int32)

    pltpu.sync_copy(mask_hbm.at[pl.ds(base, CHUNK)], mask_v)
    pltpu.sync_copy(lcsum_hbm.at[pl.ds(base, CHUNK)], lcsum_v)
    pltpu.sync_copy(inds2_hbm.at[pl.ds(wid * NSUB, NSUB)], inds2_v)
    pltpu.sync_copy(totals_hbm, tot_v)

    t_lo = plsc.load_gather(tot_v, [lanes, zeros16])
    t_hi = plsc.load_gather(tot_v, [lanes + 16, zeros16])
    zero_v = jnp.zeros((_LANES,), jnp.int32)
    my_pre = (jnp.sum(jnp.where(lanes < wid, t_lo, zero_v))
              + jnp.sum(jnp.where(lanes + 16 < wid, t_hi, zero_v)))
    tot_all = jnp.sum(t_lo) + jnp.sum(t_hi)

    @pl.loop(0, CHUNK // _LANES)
    def _dest(k):
        m = mask_v[pl.ds(k * _LANES, _LANES)]
        cs = lcsum_v[pl.ds(k * _LANES, _LANES)]
        gi = cs + my_pre
        ivec = base + k * _LANES + lanes
        d = jnp.where(m == 1, gi - 1, tot_all + ivec - gi)
        dest2_v[k >> 3, pl.ds((k & 7) * _LANES, _LANES)] = d

    def fire(j, b):
        pltpu.async_copy(vfeats_hbm.at[inds2_v.at[j]], feats_bufs[b], gsems[b])
        pltpu.async_copy(points_hbm.at[pl.ds(base + j * SUB, SUB)],
                         pts_bufs[b], gsems[b])
        pltpu.async_copy(coors_hbm.at[pl.ds(base + j * SUB, SUB)],
                         coor_bufs[b], gsems[b])

    def drain_gathers(b):
        pltpu.make_async_copy(vfeats_hbm.at[inds2_v.at[0]], feats_bufs[b],
                              gsems[b]).wait()
        pltpu.make_async_copy(points_hbm.at[pl.ds(0, SUB)], pts_bufs[b],
                              gsems[b]).wait()
        pltpu.make_async_copy(coors_hbm.at[pl.ds(0, SUB)], coor_bufs[b],
                              gsems[b]).wait()

    def drain_scatter(b):
        pltpu.make_async_copy(out_bufs[b], out_hbm.at[dest2_v.at[0]],
                              ssems[b]).wait()

    def compute(j, b):
        feats_buf = feats_bufs[b]
        pts_buf = pts_bufs[b]
        coor_buf = coor_bufs[b]
        out_buf = out_bufs[b]

        @pl.loop(0, SUB)
        def _copy(r):
            for cc in range(C // _LANES):
                out_buf[r, pl.ds(cc * _LANES, _LANES)] = (
                    feats_buf[r, pl.ds(cc * _LANES, _LANES)])

        for grp in range(SUB // _LANES):
            rows = grp * _LANES + lanes
            px = plsc.load_gather(pts_buf, [rows, zeros16])
            py = plsc.load_gather(pts_buf, [rows, zeros16 + 1])
            pz = plsc.load_gather(pts_buf, [rows, zeros16 + 2])
            c3 = plsc.load_gather(coor_buf, [rows, zeros16 + 3])
            c2 = plsc.load_gather(coor_buf, [rows, zeros16 + 2])
            c1 = plsc.load_gather(coor_buf, [rows, zeros16 + 1])
            cx = (c3.astype(jnp.float32) + 0.5) * VOXEL_SIZE[0] + PC_MIN[0]
            cy = (c2.astype(jnp.float32) + 0.5) * VOXEL_SIZE[1] + PC_MIN[1]
            cz = (c1.astype(jnp.float32) + 0.5) * VOXEL_SIZE[2] + PC_MIN[2]
            plsc.store_scatter(out_buf, [rows, zeros16 + C], px - cx)
            plsc.store_scatter(out_buf, [rows, zeros16 + C + 1], py - cy)
            plsc.store_scatter(out_buf, [rows, zeros16 + C + 2], pz - cz)

    fire(0, 0)
    fire(1, 1)

    @pl.loop(0, NSUB, step=2)
    def _ring(g):
        for b in range(2):
            j = g + b

            @pl.when(g >= 2)
            def _():
                drain_scatter(b)

            drain_gathers(b)
            compute(j, b)
            pltpu.async_copy(out_bufs[b], out_hbm.at[dest2_v.at[j]], ssems[b])

            @pl.when(j + 2 < NSUB)
            def _():
                fire(j + 2, b)

    drain_scatter(0)
    drain_scatter(1)


def _main_pass(points, pts_coors, voxel_feats, inds2, mask, lcsum, totals):
    mesh = plsc.VectorSubcoreMesh(core_axis_name="c", subcore_axis_name="s")
    return pl.kernel(
        _main_body,
        compiler_params=pltpu.CompilerParams(needs_layout_passes=False, use_tc_tiling_on_sc=False),
        out_type=jax.ShapeDtypeStruct((N, C + 3), jnp.float32),
        mesh=mesh,
        scratch_types=[
            pltpu.VMEM((CHUNK,), jnp.int32),
            pltpu.VMEM((CHUNK,), jnp.int32),
            pltpu.VMEM((NSUB, SUB), jnp.int32),
            pltpu.VMEM((NSUB, SUB), jnp.int32),
            pltpu.VMEM((NW, _LANES), jnp.int32),
            pltpu.VMEM((SUB, C), jnp.float32),
            pltpu.VMEM((SUB, C), jnp.float32),
            pltpu.VMEM((SUB, 4), jnp.float32),
            pltpu.VMEM((SUB, 4), jnp.float32),
            pltpu.VMEM((SUB, 4), jnp.int32),
            pltpu.VMEM((SUB, 4), jnp.int32),
            pltpu.VMEM((SUB, C + 3), jnp.float32),
            pltpu.VMEM((SUB, C + 3), jnp.float32),
            pltpu.SemaphoreType.DMA,
            pltpu.SemaphoreType.DMA,
            pltpu.SemaphoreType.DMA,
            pltpu.SemaphoreType.DMA,
        ],
    )(points, pts_coors, voxel_feats, inds2, mask, lcsum, totals)


def kernel(points, pts_coors, voxel_feats, voxel2point_inds, voxel_padding):
    pad = jnp.asarray(voxel_padding, jnp.float32).reshape(1, 1)
    flags = _compute_flags(pad, voxel_feats)
    mask_i32, lcsum, totals = _mask_scan(flags.reshape(M // 128, 128),
                                         voxel2point_inds)
    inds2 = voxel2point_inds.reshape(N // SUB, SUB)
    results = _main_pass(points, pts_coors, voxel_feats, inds2,
                         mask_i32, lcsum, totals)
    return results, mask_i32.astype(bool)
